# Initial kernel scaffold; baseline (speedup 1.0000x reference)
import jax
import jax.numpy as jnp
from jax.experimental import pallas as pl


def kernel(x, edge_index, edge_attr, batch, W1, b1, W2, b2):
    raise NotImplementedError("write your pallas kernel here")


# speedup vs baseline: 11.3913x; 11.3913x over previous
import functools

import numpy as np
import jax
import jax.numpy as jnp
from jax import lax
from jax.experimental import pallas as pl
from jax.experimental.pallas import tpu as pltpu
from jax.experimental.pallas import tpu_sc as plsc

F32 = jnp.float32
I32 = jnp.int32

NC = 2
NS = 16
NW = NC * NS
LANES = 16
CHK = 128
NUM_GRAPHS = 8
DROP_EDGE = 0.1

_keep_cache = {}


def _keep_mask(E):
    if E not in _keep_cache:
        num_keep = int((1.0 - DROP_EDGE) * E)

        def build():
            perm = jax.random.permutation(jax.random.key(42), E)[:num_keep]
            return jnp.zeros((E,), jnp.float32).at[perm].set(1.0)

        try:
            with jax.ensure_compile_time_eval():
                m = np.asarray(build())
        except Exception:
            return build()
        _keep_cache[E] = m
    return _keep_cache[E]


def _prep_body(ea_ref, keep_ref, ew_ref, ew1_ref):
    ea = ea_ref[...]
    mx = jnp.max(ea)
    mn = jnp.min(ea)
    ew = (mx - ea) / (mx - mn)
    ew_ref[...] = ew
    ew1_ref[...] = ew * keep_ref[...]


def _scale_body(x_ref, w1_ref, d1a_ref, d1b_ref, d2a_ref, d2b_ref,
                g1_ref, dis1_ref, dis2_ref):
    n = x_ref.shape[0]
    npad = g1_ref.shape[0]
    dh = x_ref.shape[1]
    d1 = d1a_ref[...] + d1b_ref[...] + 1.0
    d2 = d2a_ref[...] + d2b_ref[...] + 1.0
    dis1 = lax.rsqrt(d1)
    dis2 = lax.rsqrt(d2)
    dis1_ref[...] = dis1
    dis2_ref[...] = dis2
    h = jnp.dot(x_ref[...], w1_ref[...], preferred_element_type=F32)
    g1 = h * dis1[:n][:, None]
    nf = h.shape[1]
    g1w = jnp.concatenate([g1, jnp.zeros((n, 128 - nf), F32)], axis=1)
    g1_ref[...] = jnp.concatenate(
        [g1w, jnp.zeros((npad - n, 128), F32)], axis=0)


def _mid_body(s1p_ref, g1_ref, dis1_ref, dis2_ref, b1_ref, g2_ref):
    npad = g1_ref.shape[0]
    nf = b1_ref.shape[0]
    s1 = s1p_ref[0, :, :nf] + s1p_ref[1, :, :nf]
    g1 = g1_ref[..., :nf]
    pre = dis1_ref[...][:, None] * (s1 + g1) + b1_ref[...][None, :]
    x1 = jnp.maximum(pre, 0.0)
    g2 = x1 * dis2_ref[...][:, None]
    g2_ref[...] = jnp.concatenate(
        [g2, jnp.zeros((npad, 128 - nf), F32)], axis=1)


def _final_body(s2p_ref, g2_ref, dis2_ref, w2_ref, b2_ref,
                x_ref, bt_ref, out_ref):
    n = x_ref.shape[0]
    nf = w2_ref.shape[0]
    s2 = s2p_ref[0, :, :nf] + s2p_ref[1, :, :nf]
    g2 = g2_ref[..., :nf]
    agg = dis2_ref[...][:, None] * (s2 + g2)
    agg = agg[:n]
    x2 = (jnp.dot(agg, w2_ref[...], preferred_element_type=F32)
          + b2_ref[...][None, :] + x_ref[...])
    bt = bt_ref[...][:n]
    neg = jnp.float32(-jnp.inf)
    outs = []
    for gidx in range(NUM_GRAPHS):
        m = bt == gidx
        outs.append(jnp.max(jnp.where(m, x2, neg), axis=0))
    out_ref[...] = jnp.stack(outs, axis=0)


def _zero_vec():
    return jnp.zeros((LANES,), F32)


@functools.lru_cache(maxsize=None)
def _deg_kernel(NP, NCH):
    rpt = NP // NS
    mesh = plsc.VectorSubcoreMesh(core_axis_name="c", subcore_axis_name="s")

    @functools.partial(
        pl.kernel,
        out_type=(jax.ShapeDtypeStruct((NP,), F32),
                  jax.ShapeDtypeStruct((NP,), F32),
                  jax.ShapeDtypeStruct((NP,), F32),
                  jax.ShapeDtypeStruct((NP,), F32)),
        mesh=mesh,
        scratch_types=[
            pltpu.VMEM((NCH, CHK), I32),
            pltpu.VMEM((NCH, CHK), F32),
            pltpu.VMEM((NCH, CHK), F32),
            pltpu.VMEM((rpt,), F32),
            pltpu.VMEM_SHARED((NP,), F32),
            pltpu.VMEM_SHARED((NP,), F32),
        ],
    )
    def deg(col_hbm, w1_hbm, w2_hbm, d1a, d1b, d2a, d2b,
            colv, w1v, w2v, zb, d1sh, d2sh):
        c = lax.axis_index("c")
        s = lax.axis_index("s")
        wid = c * NS + s

        @pl.loop(0, rpt // LANES)
        def _(i):
            zb[pl.ds(i * LANES, LANES)] = _zero_vec()

        pltpu.sync_copy(zb, d1sh.at[pl.ds(s * rpt, rpt)])
        pltpu.sync_copy(zb, d2sh.at[pl.ds(s * rpt, rpt)])
        pltpu.sync_copy(col_hbm.at[pl.ds(wid * NCH, NCH)], colv)
        pltpu.sync_copy(w1_hbm.at[pl.ds(wid * NCH, NCH)], w1v)
        pltpu.sync_copy(w2_hbm.at[pl.ds(wid * NCH, NCH)], w2v)
        plsc.subcore_barrier()

        @pl.loop(0, NCH)
        def _(j):
            pltpu.sync_copy(w1v.at[j], d1sh.at[colv.at[j]], add=True)
            pltpu.sync_copy(w2v.at[j], d2sh.at[colv.at[j]], add=True)

        plsc.subcore_barrier()
        sl = pl.ds(s * rpt, rpt)

        @pl.when(c == 0)
        def _():
            pltpu.sync_copy(d1sh.at[sl], zb)
            pltpu.sync_copy(zb, d1a.at[sl])
            pltpu.sync_copy(d2sh.at[sl], zb)
            pltpu.sync_copy(zb, d2a.at[sl])

        @pl.when(c == 1)
        def _():
            pltpu.sync_copy(d1sh.at[sl], zb)
            pltpu.sync_copy(zb, d1b.at[sl])
            pltpu.sync_copy(d2sh.at[sl], zb)
            pltpu.sync_copy(zb, d2b.at[sl])

    return deg


@functools.lru_cache(maxsize=None)
def _msg_kernel(NP, NCH):
    rpt = NP // NS
    PIECE = 16
    mesh = plsc.VectorSubcoreMesh(core_axis_name="c", subcore_axis_name="s")

    @functools.partial(
        pl.kernel,
        out_type=jax.ShapeDtypeStruct((NC, NP, 128), F32),
        mesh=mesh,
        scratch_types=[
            pltpu.VMEM((PIECE, CHK), I32),
            pltpu.VMEM((PIECE, CHK), I32),
            pltpu.VMEM((PIECE, CHK), F32),
            pltpu.VMEM((CHK, 128), F32),
            pltpu.VMEM_SHARED((NP, 128), F32),
            pltpu.SemaphoreType.DMA,
        ],
    )
    def msg_k(g_hbm, row_hbm, col_hbm, w_hbm, out_hbm,
              rowv, colv, wv, msg, ssh, gsem):
        c = lax.axis_index("c")
        s = lax.axis_index("s")
        wid = c * NS + s

        @pl.loop(0, CHK)
        def _(i):
            for d in range(8):
                msg[i, pl.ds(d * LANES, LANES)] = _zero_vec()

        nfull = rpt // CHK
        for i in range(nfull):
            pltpu.sync_copy(msg, ssh.at[pl.ds(s * rpt + i * CHK, CHK)])
        rem = rpt - nfull * CHK
        if rem:
            pltpu.sync_copy(msg.at[pl.ds(0, rem), :],
                            ssh.at[pl.ds(s * rpt + nfull * CHK, rem)])
        plsc.subcore_barrier()

        @pl.loop(0, NCH // PIECE)
        def _(p):
            base = wid * NCH + p * PIECE
            pltpu.sync_copy(row_hbm.at[pl.ds(base, PIECE)], rowv)
            pltpu.sync_copy(col_hbm.at[pl.ds(base, PIECE)], colv)
            pltpu.sync_copy(w_hbm.at[pl.ds(base, PIECE)], wv)

            @pl.loop(0, PIECE)
            def _(j):
                pltpu.async_copy(g_hbm.at[rowv.at[j]], msg, gsem).wait()

                @pl.loop(0, CHK // LANES)
                def _(eg):
                    w16 = wv[j, pl.ds(eg * LANES, LANES)]
                    for k in range(LANES):
                        wb = jnp.full((LANES,), w16[k], F32)
                        e = eg * LANES + k
                        msg[e, pl.ds(0, LANES)] = msg[e, pl.ds(0, LANES)] * wb
                        msg[e, pl.ds(LANES, LANES)] = (
                            msg[e, pl.ds(LANES, LANES)] * wb)

                pltpu.sync_copy(msg, ssh.at[colv.at[j]], add=True)

        plsc.subcore_barrier()
        for i in range(nfull):
            sl = pl.ds(s * rpt + i * CHK, CHK)
            pltpu.sync_copy(ssh.at[sl], msg)
            pltpu.sync_copy(msg, out_hbm.at[c, sl])
        if rem:
            sl = pl.ds(s * rpt + nfull * CHK, rem)
            pltpu.sync_copy(ssh.at[sl], msg.at[pl.ds(0, rem), :])
            pltpu.sync_copy(msg.at[pl.ds(0, rem), :], out_hbm.at[c, sl])

    return msg_k


def kernel(x, edge_index, edge_attr, batch, W1, b1, W2, b2):
    N, DIN = x.shape
    E = edge_index.shape[1]
    DH = W1.shape[1]
    DOUT = W2.shape[1]
    NP = ((N + 127) // 128) * 128
    NCH = -(-E // (NW * CHK))
    NCH = ((NCH + 7) // 8) * 8
    EP = NW * CHK * NCH

    ea = edge_attr[:, 1]
    keep = jnp.asarray(_keep_mask(E))

    ew, ew1 = pl.pallas_call(
        _prep_body,
        out_shape=(jax.ShapeDtypeStruct((E // 128, 128), F32),
                   jax.ShapeDtypeStruct((E // 128, 128), F32)),
    )(ea.reshape(E // 128, 128), keep.reshape(E // 128, 128))

    zpad = jnp.zeros((EP - E,), F32)
    ipad = jnp.zeros((EP - E,), I32)
    row2 = jnp.concatenate([edge_index[0], ipad]).reshape(EP // CHK, CHK)
    col2 = jnp.concatenate([edge_index[1], ipad]).reshape(EP // CHK, CHK)
    ew2d = jnp.concatenate([ew.reshape(E), zpad]).reshape(EP // CHK, CHK)
    ew1d = jnp.concatenate([ew1.reshape(E), zpad]).reshape(EP // CHK, CHK)

    d1a, d1b, d2a, d2b = _deg_kernel(NP, NCH)(col2, ew1d, ew2d)

    g1, dis1, dis2 = pl.pallas_call(
        _scale_body,
        out_shape=(jax.ShapeDtypeStruct((NP, 128), F32),
                   jax.ShapeDtypeStruct((NP,), F32),
                   jax.ShapeDtypeStruct((NP,), F32)),
    )(x, W1, d1a, d1b, d2a, d2b)

    s1p = _msg_kernel(NP, NCH)(g1, row2, col2, ew1d)

    g2 = pl.pallas_call(
        _mid_body,
        out_shape=jax.ShapeDtypeStruct((NP, 128), F32),
    )(s1p, g1, dis1, dis2, b1)

    s2p = _msg_kernel(NP, NCH)(g2, row2, col2, ew2d)

    batch_pad = jnp.concatenate(
        [batch, jnp.full((NP - N,), NUM_GRAPHS, batch.dtype)]).reshape(NP, 1)

    out = pl.pallas_call(
        _final_body,
        out_shape=jax.ShapeDtypeStruct((NUM_GRAPHS, DOUT), F32),
    )(s2p, g2, dis2, W2, b2, x, batch_pad)

    return out


# speedup vs baseline: 11.7407x; 1.0307x over previous
import functools

import numpy as np
import jax
import jax.numpy as jnp
from jax import lax
from jax.experimental import pallas as pl
from jax.experimental.pallas import tpu as pltpu
from jax.experimental.pallas import tpu_sc as plsc

F32 = jnp.float32
I32 = jnp.int32

NC = 2
NS = 16
NW = NC * NS
LANES = 16
CHK = 80
NUM_GRAPHS = 8
DROP_EDGE = 0.1

_keep_cache = {}


def _keep_mask(E):
    if E not in _keep_cache:
        num_keep = int((1.0 - DROP_EDGE) * E)

        def build():
            perm = jax.random.permutation(jax.random.key(42), E)[:num_keep]
            return jnp.zeros((E,), jnp.float32).at[perm].set(1.0)

        try:
            with jax.ensure_compile_time_eval():
                m = np.asarray(build())
        except Exception:
            return build()
        _keep_cache[E] = m
    return _keep_cache[E]


def _prep_body(ea_ref, keep_ref, ew_ref, ew1_ref):
    ea = ea_ref[...]
    mx = jnp.max(ea)
    mn = jnp.min(ea)
    ew = (mx - ea) / (mx - mn)
    ew_ref[...] = ew
    ew1_ref[...] = ew * keep_ref[...]


def _scale_body(x_ref, w1_ref, d1a_ref, d1b_ref, d2a_ref, d2b_ref,
                g1_ref, dis1_ref, dis2_ref):
    n = x_ref.shape[0]
    npad = g1_ref.shape[0]
    dh = x_ref.shape[1]
    d1 = d1a_ref[...] + d1b_ref[...] + 1.0
    d2 = d2a_ref[...] + d2b_ref[...] + 1.0
    dis1 = lax.rsqrt(d1)
    dis2 = lax.rsqrt(d2)
    dis1_ref[...] = dis1
    dis2_ref[...] = dis2
    h = jnp.dot(x_ref[...], w1_ref[...], preferred_element_type=F32)
    g1 = h * dis1[:n][:, None]
    nf = h.shape[1]
    g1w = jnp.concatenate([g1, jnp.zeros((n, 128 - nf), F32)], axis=1)
    g1_ref[...] = jnp.concatenate(
        [g1w, jnp.zeros((npad - n, 128), F32)], axis=0)


def _mid_body(s1p_ref, g1_ref, dis1_ref, dis2_ref, b1_ref, g2_ref):
    npad = g1_ref.shape[0]
    nf = b1_ref.shape[0]
    s1 = s1p_ref[0, :, :nf] + s1p_ref[1, :, :nf]
    g1 = g1_ref[..., :nf]
    pre = dis1_ref[...][:, None] * (s1 + g1) + b1_ref[...][None, :]
    x1 = jnp.maximum(pre, 0.0)
    g2 = x1 * dis2_ref[...][:, None]
    g2_ref[...] = jnp.concatenate(
        [g2, jnp.zeros((npad, 128 - nf), F32)], axis=1)


def _final_body(s2p_ref, g2_ref, dis2_ref, w2_ref, b2_ref,
                x_ref, bt_ref, out_ref):
    n = x_ref.shape[0]
    nf = w2_ref.shape[0]
    s2 = s2p_ref[0, :, :nf] + s2p_ref[1, :, :nf]
    g2 = g2_ref[..., :nf]
    agg = dis2_ref[...][:, None] * (s2 + g2)
    agg = agg[:n]
    x2 = (jnp.dot(agg, w2_ref[...], preferred_element_type=F32)
          + b2_ref[...][None, :] + x_ref[...])
    bt = bt_ref[...][:n]
    neg = jnp.float32(-jnp.inf)
    outs = []
    for gidx in range(NUM_GRAPHS):
        m = bt == gidx
        outs.append(jnp.max(jnp.where(m, x2, neg), axis=0))
    out_ref[...] = jnp.stack(outs, axis=0)


def _zero_vec():
    return jnp.zeros((LANES,), F32)


@functools.lru_cache(maxsize=None)
def _deg_kernel(NP, NCH):
    rpt = NP // NS
    mesh = plsc.VectorSubcoreMesh(core_axis_name="c", subcore_axis_name="s")

    @functools.partial(
        pl.kernel,
        out_type=(jax.ShapeDtypeStruct((NP,), F32),
                  jax.ShapeDtypeStruct((NP,), F32),
                  jax.ShapeDtypeStruct((NP,), F32),
                  jax.ShapeDtypeStruct((NP,), F32)),
        mesh=mesh,
        scratch_types=[
            pltpu.VMEM((NCH, CHK), I32),
            pltpu.VMEM((NCH, CHK), F32),
            pltpu.VMEM((NCH, CHK), F32),
            pltpu.VMEM((rpt,), F32),
            pltpu.VMEM_SHARED((NP,), F32),
            pltpu.VMEM_SHARED((NP,), F32),
        ],
    )
    def deg(col_hbm, w1_hbm, w2_hbm, d1a, d1b, d2a, d2b,
            colv, w1v, w2v, zb, d1sh, d2sh):
        c = lax.axis_index("c")
        s = lax.axis_index("s")
        wid = c * NS + s

        @pl.loop(0, rpt // LANES)
        def _(i):
            zb[pl.ds(i * LANES, LANES)] = _zero_vec()

        pltpu.sync_copy(zb, d1sh.at[pl.ds(s * rpt, rpt)])
        pltpu.sync_copy(zb, d2sh.at[pl.ds(s * rpt, rpt)])
        pltpu.sync_copy(col_hbm.at[pl.ds(wid * NCH, NCH)], colv)
        pltpu.sync_copy(w1_hbm.at[pl.ds(wid * NCH, NCH)], w1v)
        pltpu.sync_copy(w2_hbm.at[pl.ds(wid * NCH, NCH)], w2v)
        plsc.subcore_barrier()

        @pl.loop(0, NCH)
        def _(j):
            pltpu.sync_copy(w1v.at[j], d1sh.at[colv.at[j]], add=True)
            pltpu.sync_copy(w2v.at[j], d2sh.at[colv.at[j]], add=True)

        plsc.subcore_barrier()
        sl = pl.ds(s * rpt, rpt)

        @pl.when(c == 0)
        def _():
            pltpu.sync_copy(d1sh.at[sl], zb)
            pltpu.sync_copy(zb, d1a.at[sl])
            pltpu.sync_copy(d2sh.at[sl], zb)
            pltpu.sync_copy(zb, d2a.at[sl])

        @pl.when(c == 1)
        def _():
            pltpu.sync_copy(d1sh.at[sl], zb)
            pltpu.sync_copy(zb, d1b.at[sl])
            pltpu.sync_copy(d2sh.at[sl], zb)
            pltpu.sync_copy(zb, d2b.at[sl])

    return deg


@functools.lru_cache(maxsize=None)
def _msg_kernel(NP, NCH):
    rpt = NP // NS
    PIECE = 16
    R = 4
    mesh = plsc.VectorSubcoreMesh(core_axis_name="c", subcore_axis_name="s")

    @functools.partial(
        pl.kernel,
        out_type=jax.ShapeDtypeStruct((NC, NP, 128), F32),
        mesh=mesh,
        scratch_types=[
            pltpu.VMEM((PIECE, CHK), I32),
            pltpu.VMEM((PIECE, CHK), I32),
            pltpu.VMEM((PIECE, CHK), F32),
        ] + [pltpu.VMEM((CHK, 128), F32)] * R + [
            pltpu.VMEM_SHARED((NP, 128), F32),
        ] + [pltpu.SemaphoreType.DMA] * (2 * R),
    )
    def msg_k(g_hbm, row_hbm, col_hbm, w_hbm, out_hbm,
              rowv, colv, wv, m0, m1, m2, m3, ssh,
              g0, g1, g2, g3, s0, s1, s2, s3):
        c = lax.axis_index("c")
        s = lax.axis_index("s")
        wid = c * NS + s
        msgs = [m0, m1, m2, m3]
        gsem = [g0, g1, g2, g3]
        ssem = [s0, s1, s2, s3]

        @pl.loop(0, CHK)
        def _(i):
            for d in range(8):
                m0[i, pl.ds(d * LANES, LANES)] = _zero_vec()

        nfull = rpt // CHK
        rem = rpt - nfull * CHK
        for i in range(nfull):
            pltpu.sync_copy(m0, ssh.at[pl.ds(s * rpt + i * CHK, CHK)])
        if rem:
            pltpu.sync_copy(m0.at[pl.ds(0, rem), :],
                            ssh.at[pl.ds(s * rpt + nfull * CHK, rem)])
        plsc.subcore_barrier()

        def start_gather(r, j):
            pltpu.async_copy(g_hbm.at[rowv.at[j]], msgs[r], gsem[r])

        def wait_gather(r, j):
            pltpu.make_async_copy(g_hbm.at[rowv.at[j]], msgs[r],
                                  gsem[r]).wait()

        def start_scatter(r, j):
            pltpu.async_copy(msgs[r], ssh.at[colv.at[j]], ssem[r], add=True)

        def wait_scatter(r):
            pltpu.make_async_copy(msgs[r], ssh.at[colv.at[0]],
                                  ssem[r]).wait()

        @pl.loop(0, NCH // PIECE)
        def _(p):
            base = wid * NCH + p * PIECE
            pltpu.sync_copy(row_hbm.at[pl.ds(base, PIECE)], rowv)
            pltpu.sync_copy(col_hbm.at[pl.ds(base, PIECE)], colv)
            pltpu.sync_copy(w_hbm.at[pl.ds(base, PIECE)], wv)
            start_gather(0, 0)
            start_gather(1, 1)

            @pl.loop(0, PIECE // R)
            def _(q):
                for rr in range(R):
                    j = q * R + rr
                    wait_gather(rr, j)

                    @pl.loop(0, CHK // LANES)
                    def _(eg):
                        w16 = wv[j, pl.ds(eg * LANES, LANES)]
                        for k in range(LANES):
                            wb = jnp.full((LANES,), w16[k], F32)
                            e = eg * LANES + k
                            sl0 = pl.ds(0, LANES)
                            sl1 = pl.ds(LANES, LANES)
                            msgs[rr][e, sl0] = msgs[rr][e, sl0] * wb
                            msgs[rr][e, sl1] = msgs[rr][e, sl1] * wb

                    start_scatter(rr, j)
                    nj = j + 2
                    rn = (rr + 2) % R

                    @pl.when((nj >= R) & (nj < PIECE))
                    def _():
                        wait_scatter(rn)

                    @pl.when(nj < PIECE)
                    def _():
                        start_gather(rn, nj)

            for rr in range(R):
                wait_scatter(rr)

        plsc.subcore_barrier()
        for i in range(nfull):
            sl = pl.ds(s * rpt + i * CHK, CHK)
            pltpu.sync_copy(ssh.at[sl], m0)
            pltpu.sync_copy(m0, out_hbm.at[c, sl])
        if rem:
            sl = pl.ds(s * rpt + nfull * CHK, rem)
            pltpu.sync_copy(ssh.at[sl], m0.at[pl.ds(0, rem), :])
            pltpu.sync_copy(m0.at[pl.ds(0, rem), :], out_hbm.at[c, sl])

    return msg_k


def kernel(x, edge_index, edge_attr, batch, W1, b1, W2, b2):
    N, DIN = x.shape
    E = edge_index.shape[1]
    DH = W1.shape[1]
    DOUT = W2.shape[1]
    NP = ((N + 127) // 128) * 128
    NCH = -(-E // (NW * CHK))
    NCH = ((NCH + 7) // 8) * 8
    EP = NW * CHK * NCH

    ea = edge_attr[:, 1]
    keep = jnp.asarray(_keep_mask(E))

    ew, ew1 = pl.pallas_call(
        _prep_body,
        out_shape=(jax.ShapeDtypeStruct((E // 128, 128), F32),
                   jax.ShapeDtypeStruct((E // 128, 128), F32)),
    )(ea.reshape(E // 128, 128), keep.reshape(E // 128, 128))

    zpad = jnp.zeros((EP - E,), F32)
    ipad = jnp.zeros((EP - E,), I32)
    row2 = jnp.concatenate([edge_index[0], ipad]).reshape(EP // CHK, CHK)
    col2 = jnp.concatenate([edge_index[1], ipad]).reshape(EP // CHK, CHK)
    ew2d = jnp.concatenate([ew.reshape(E), zpad]).reshape(EP // CHK, CHK)
    ew1d = jnp.concatenate([ew1.reshape(E), zpad]).reshape(EP // CHK, CHK)

    d1a, d1b, d2a, d2b = _deg_kernel(NP, NCH)(col2, ew1d, ew2d)

    g1, dis1, dis2 = pl.pallas_call(
        _scale_body,
        out_shape=(jax.ShapeDtypeStruct((NP, 128), F32),
                   jax.ShapeDtypeStruct((NP,), F32),
                   jax.ShapeDtypeStruct((NP,), F32)),
    )(x, W1, d1a, d1b, d2a, d2b)

    s1p = _msg_kernel(NP, NCH)(g1, row2, col2, ew1d)

    g2 = pl.pallas_call(
        _mid_body,
        out_shape=jax.ShapeDtypeStruct((NP, 128), F32),
    )(s1p, g1, dis1, dis2, b1)

    s2p = _msg_kernel(NP, NCH)(g2, row2, col2, ew2d)

    batch_pad = jnp.concatenate(
        [batch, jnp.full((NP - N,), NUM_GRAPHS, batch.dtype)]).reshape(NP, 1)

    out = pl.pallas_call(
        _final_body,
        out_shape=jax.ShapeDtypeStruct((NUM_GRAPHS, DOUT), F32),
    )(s2p, g2, dis2, W2, b2, x, batch_pad)

    return out
